# baseline (speedup 1.0000x reference)
import math

import jax
import jax.numpy as jnp
from jax import lax
from jax.experimental import pallas as pl
from jax.experimental.pallas import tpu as pltpu
from jax.experimental.pallas import tpu_sc as plsc

D = 1024
B_N = 4
S_N = 2048
NTOK = B_N * S_N
NC, NS, L = 2, 16, 16
NW = NC * NS
S_PER_W = S_N // NW
C = 16
NQ = S_PER_W // C
NCHUNK = NQ * B_N
NBUF = 5
SCALE = math.sqrt(D)


def _body(x_hbm, tok_hbm, pos_hbm, out_hbm,
          idx_v, g0_v, g1_v, g2_v, g3_v, g4_v, p0_v, p1_v, isem,
          gsem0, gsem1, gsem2, gsem3, gsem4,
          psem0, psem1, ssem0, ssem1, ssem2, ssem3, ssem4):
    wid = lax.axis_index("s") * NC + lax.axis_index("c")
    s0 = wid * S_PER_W

    g_bufs = (g0_v, g1_v, g2_v, g3_v, g4_v)
    gsems = (gsem0, gsem1, gsem2, gsem3, gsem4)
    ssems = (ssem0, ssem1, ssem2, ssem3, ssem4)
    p_bufs = (p0_v, p1_v)
    psems = (psem0, psem1)

    def chunk_row0(c):
        return (c % B_N) * S_N + s0 + (c // B_N) * C

    def idx_off(c):
        return (c % B_N) * S_PER_W + (c // B_N) * C

    def gather(c):
        k = c % NBUF
        h = C // 2
        a = pltpu.async_copy(
            tok_hbm.at[idx_v.at[pl.ds(idx_off(c), h)]],
            g_bufs[k].at[pl.ds(0, h)], gsems[k])
        b = pltpu.async_copy(
            tok_hbm.at[idx_v.at[pl.ds(idx_off(c) + h, h)]],
            g_bufs[k].at[pl.ds(h, h)], gsems[k])
        return (a, b)

    def pos_load(q):
        return pltpu.async_copy(
            pos_hbm.at[pl.ds(s0 + q * C, C)], p_bufs[q % 2], psems[q % 2])

    def compute(q):
        gs = tuple(g_bufs[(B_N * q + b) % NBUF] for b in range(B_N))
        p_v = p_bufs[q % 2]

        def fuse_row(r, _):
            for j in range(D // L):
                sl = pl.ds(j * L, L)
                p = p_v[r, sl]
                for b in range(B_N):
                    gs[b][r, sl] = gs[b][r, sl] * SCALE + p
            return 0
        lax.fori_loop(0, C, fuse_row, 0)

    idx_copies = [
        pltpu.async_copy(x_hbm.at[pl.ds(b * S_N + s0, S_PER_W)],
                         idx_v.at[pl.ds(b * S_PER_W, S_PER_W)], isem)
        for b in range(B_N)
    ]
    pos_loads = {0: pos_load(0), 1: pos_load(1)}
    for cp in idx_copies:
        cp.wait()
    gathers = {c: gather(c) for c in range(NBUF)}
    stores = {}

    for q in range(NQ):
        for b in range(B_N):
            for cp in gathers.pop(B_N * q + b):
                cp.wait()
        if q in pos_loads:
            pos_loads.pop(q).wait()
        compute(q)
        if q + 2 < NQ:
            pos_loads[q + 2] = pos_load(q + 2)
        for b in range(B_N):
            c = B_N * q + b
            stores[c] = pltpu.async_copy(
                g_bufs[c % NBUF], out_hbm.at[pl.ds(chunk_row0(c), C)],
                ssems[c % NBUF])
        for c in range(B_N * q + NBUF, min(B_N * q + NBUF + B_N, NCHUNK)):
            stores.pop(c - NBUF).wait()
            gathers[c] = gather(c)

    for st in stores.values():
        st.wait()


@jax.jit
def _embed(x_flat, token_table, pos_table):
    mesh = plsc.VectorSubcoreMesh(
        core_axis_name="c", subcore_axis_name="s", num_cores=NC, num_subcores=NS
    )
    run = pl.kernel(
        _body,
        out_type=jax.ShapeDtypeStruct((NTOK, D), jnp.float32),
        mesh=mesh,
        scratch_types=[
            pltpu.VMEM((B_N * S_PER_W,), jnp.int32),
        ] + [pltpu.VMEM((C, D), jnp.float32)] * (NBUF + 2)
          + [pltpu.SemaphoreType.DMA] * 13,
    )
    return run(x_flat, token_table, pos_table)


def kernel(x, token_table, pos_table):
    x_flat = x.reshape(-1).astype(jnp.int32)
    out = _embed(x_flat, token_table, pos_table)
    return out.reshape(B_N, S_N, D)


# speedup vs baseline: 1.0072x; 1.0072x over previous
import math

import jax
import jax.numpy as jnp
from jax import lax
from jax.experimental import pallas as pl
from jax.experimental.pallas import tpu as pltpu
from jax.experimental.pallas import tpu_sc as plsc

D = 1024
B_N = 4
S_N = 2048
NTOK = B_N * S_N
NC, NS, L = 2, 16, 16
NW = NC * NS
S_PER_W = S_N // NW
C = 16
NQ = S_PER_W // C
NCHUNK = NQ * B_N
NBUF = 5
SCALE = math.sqrt(D)


def _body(x_hbm, tok_hbm, pos_hbm, out_hbm,
          idx_v, g0_v, g1_v, g2_v, g3_v, g4_v, p0_v, p1_v, isem,
          gsem0, gsem1, gsem2, gsem3, gsem4,
          psem0, psem1, ssem0, ssem1, ssem2, ssem3, ssem4):
    wid = lax.axis_index("s") * NC + lax.axis_index("c")
    s0 = wid * S_PER_W

    g_bufs = (g0_v, g1_v, g2_v, g3_v, g4_v)
    gsems = (gsem0, gsem1, gsem2, gsem3, gsem4)
    ssems = (ssem0, ssem1, ssem2, ssem3, ssem4)
    p_bufs = (p0_v, p1_v)
    psems = (psem0, psem1)

    def chunk_row0(c):
        return (c % B_N) * S_N + s0 + (c // B_N) * C

    def idx_off(c):
        return (c % B_N) * S_PER_W + (c // B_N) * C

    def gather(c):
        k = c % NBUF
        return pltpu.async_copy(
            tok_hbm.at[idx_v.at[pl.ds(idx_off(c), C)]], g_bufs[k], gsems[k])

    def pos_load(q):
        return pltpu.async_copy(
            pos_hbm.at[pl.ds(s0 + q * C, C)], p_bufs[q % 2], psems[q % 2])

    def compute(q):
        gs = tuple(g_bufs[(B_N * q + b) % NBUF] for b in range(B_N))
        p_v = p_bufs[q % 2]

        def fuse_row(r, _):
            for j in range(D // L):
                sl = pl.ds(j * L, L)
                p = p_v[r, sl]
                for b in range(B_N):
                    gs[b][r, sl] = gs[b][r, sl] * SCALE + p
            return 0
        lax.fori_loop(0, C, fuse_row, 0)

    idx_copies = [
        pltpu.async_copy(x_hbm.at[pl.ds(b * S_N + s0, S_PER_W)],
                         idx_v.at[pl.ds(b * S_PER_W, S_PER_W)], isem)
        for b in range(B_N)
    ]
    pos_loads = {0: pos_load(0), 1: pos_load(1)}
    for cp in idx_copies:
        cp.wait()
    gathers = {c: gather(c) for c in range(NBUF)}
    stores = {}

    for q in range(NQ):
        for b in range(B_N):
            gathers.pop(B_N * q + b).wait()
        if q in pos_loads:
            pos_loads.pop(q).wait()
        compute(q)
        if q + 2 < NQ:
            pos_loads[q + 2] = pos_load(q + 2)
        for b in range(B_N):
            c = B_N * q + b
            stores[c] = pltpu.async_copy(
                g_bufs[c % NBUF], out_hbm.at[pl.ds(chunk_row0(c), C)],
                ssems[c % NBUF])
        for c in range(B_N * q + NBUF, min(B_N * q + NBUF + B_N, NCHUNK)):
            stores.pop(c - NBUF).wait()
            gathers[c] = gather(c)

    for st in stores.values():
        st.wait()


@jax.jit
def _embed(x_flat, token_table, pos_table):
    mesh = plsc.VectorSubcoreMesh(
        core_axis_name="c", subcore_axis_name="s", num_cores=NC, num_subcores=NS
    )
    run = pl.kernel(
        _body,
        out_type=jax.ShapeDtypeStruct((NTOK, D), jnp.float32),
        mesh=mesh,
        scratch_types=[
            pltpu.VMEM((B_N * S_PER_W,), jnp.int32),
        ] + [pltpu.VMEM((C, D), jnp.float32)] * (NBUF + 2)
          + [pltpu.SemaphoreType.DMA] * 13,
    )
    return run(x_flat, token_table, pos_table)


def kernel(x, token_table, pos_table):
    x_flat = x.reshape(-1).astype(jnp.int32)
    out = _embed(x_flat, token_table, pos_table)
    return out.reshape(B_N, S_N, D)
